# baseline (speedup 1.0000x reference)
import functools

import jax
import jax.numpy as jnp
from jax import lax
from jax.experimental import pallas as pl
from jax.experimental.pallas import tpu as pltpu
from jax.experimental.pallas import tpu_sc as plsc

_CHUNK = 16
_NBUF = 2


def kernel(emb, W):
    n_ctx, n_embd = W.shape
    seq_len = emb.shape[1]
    nw = 32
    rows_per_w = seq_len // nw
    n = rows_per_w // _CHUNK
    mesh = plsc.VectorSubcoreMesh(core_axis_name="c", subcore_axis_name="s")

    @functools.partial(
        pl.kernel,
        mesh=mesh,
        out_type=jax.ShapeDtypeStruct((seq_len, n_embd), jnp.float32),
        scratch_types=[
            pltpu.VMEM((_NBUF, _CHUNK, n_embd), jnp.float32),
            pltpu.SemaphoreType.DMA((_NBUF,)),
            pltpu.SemaphoreType.DMA((_NBUF,)),
        ],
    )
    def sc_copy(w_hbm, o_hbm, bufs, lsems, ssems):
        wid = lax.axis_index("s") * 2 + lax.axis_index("c")
        base = wid * rows_per_w

        def load(i):
            b = i % _NBUF
            return pltpu.make_async_copy(
                w_hbm.at[pl.ds(base + i * _CHUNK, _CHUNK)],
                bufs.at[b], lsems.at[b])

        def store(i):
            b = i % _NBUF
            return pltpu.make_async_copy(
                bufs.at[b],
                o_hbm.at[pl.ds(base + i * _CHUNK, _CHUNK)], ssems.at[b])

        waited = [False] * n
        for i in range(min(_NBUF - 1, n)):
            load(i).start()
        for i in range(n):
            load(i).wait()
            store(i).start()
            j = i + _NBUF - 1
            if j < n:
                if i >= 1 and not waited[i - 1]:
                    store(i - 1).wait()
                    waited[i - 1] = True
                load(j).start()
        for i in range(n):
            if not waited[i]:
                store(i).wait()

    return sc_copy(W)[None, :, :]


# speedup vs baseline: 1.0310x; 1.0310x over previous
import functools

import jax
import jax.numpy as jnp
from jax import lax
from jax.experimental import pallas as pl
from jax.experimental.pallas import tpu as pltpu
from jax.experimental.pallas import tpu_sc as plsc


def kernel(emb, W):
    n_ctx, n_embd = W.shape
    seq_len = emb.shape[1]
    nw = 32
    rows_per_w = seq_len // nw
    chunk = 16
    nchunks = rows_per_w // chunk
    mesh = plsc.VectorSubcoreMesh(core_axis_name="c", subcore_axis_name="s")

    @functools.partial(
        pl.kernel,
        mesh=mesh,
        out_type=jax.ShapeDtypeStruct((seq_len, n_embd), jnp.float32),
        scratch_types=[
            pltpu.VMEM((chunk, n_embd), jnp.float32),
            pltpu.VMEM((chunk, n_embd), jnp.float32),
            pltpu.SemaphoreType.DMA,
            pltpu.SemaphoreType.DMA,
            pltpu.SemaphoreType.DMA,
            pltpu.SemaphoreType.DMA,
        ],
    )
    def sc_copy(w_hbm, o_hbm, buf0, buf1, ls0, ls1, ss0, ss1):
        wid = lax.axis_index("s") * 2 + lax.axis_index("c")
        base = wid * rows_per_w
        bufs = (buf0, buf1)
        lsems = (ls0, ls1)
        ssems = (ss0, ss1)
        loads = [None, None]
        stores = [None, None]
        loads[0] = pltpu.async_copy(w_hbm.at[pl.ds(base, chunk)], buf0, ls0)
        for i in range(nchunks):
            b = i & 1
            nb = (i + 1) & 1
            if i + 1 < nchunks:
                if stores[nb] is not None:
                    stores[nb].wait()
                loads[nb] = pltpu.async_copy(
                    w_hbm.at[pl.ds(base + (i + 1) * chunk, chunk)],
                    bufs[nb], lsems[nb])
            loads[b].wait()
            stores[b] = pltpu.async_copy(
                bufs[b], o_hbm.at[pl.ds(base + i * chunk, chunk)], ssems[b])
        stores[0].wait()
        stores[1].wait()

    return sc_copy(W)[None, :, :]
